# baseline (speedup 1.0000x reference)
import functools

import jax
import jax.numpy as jnp
from jax import lax
from jax.experimental import pallas as pl
from jax.experimental.pallas import tpu as pltpu
from jax.experimental.pallas import tpu_sc as plsc

_APP_COUNT = 100000
_ENTITY_COUNT = 1000000
_EMB_DIM = 64
_BATCH = 16384

_NC = 2
_NS = 16
_NW = _NC * _NS
_ROWS_PER_W = _BATCH // _NW
_CHUNK = 128
_NCHUNK = _ROWS_PER_W // _CHUNK
_IDX_COLS = 128


def _sc_scores(pos_app, pos_entity, neg_app, neg_entity, app_emb, entity_emb):
  mesh = plsc.VectorSubcoreMesh(core_axis_name="c", subcore_axis_name="s")

  @functools.partial(
      pl.kernel,
      out_type=(
          jax.ShapeDtypeStruct((_BATCH,), jnp.float32),
          jax.ShapeDtypeStruct((_BATCH,), jnp.float32),
      ),
      mesh=mesh,
      compiler_params=pltpu.CompilerParams(
          needs_layout_passes=False, use_tc_tiling_on_sc=False),
      scratch_types=[
          pltpu.VMEM((_NCHUNK, _IDX_COLS), jnp.int32),
          pltpu.VMEM((_NCHUNK, _IDX_COLS), jnp.int32),
          pltpu.VMEM((_ROWS_PER_W, _EMB_DIM), jnp.float32),
          pltpu.VMEM((_ROWS_PER_W, _EMB_DIM), jnp.float32),
          pltpu.VMEM((_ROWS_PER_W,), jnp.float32),
          pltpu.SemaphoreType.DMA,
      ],
  )
  def k(pa_h, pe_h, na_h, ne_h, app_h, ent_h, pos_out, neg_out,
        idx_a, idx_e, rows_a, rows_e, svec, sem):
    wid = lax.axis_index("s") * _NC + lax.axis_index("c")
    idx_base = wid * _NCHUNK
    out_base = wid * _ROWS_PER_W

    for ia_h, ie_h, out_h in ((pa_h, pe_h, pos_out), (na_h, ne_h, neg_out)):
      pltpu.sync_copy(ia_h.at[pl.ds(idx_base, _NCHUNK)], idx_a)
      pltpu.sync_copy(ie_h.at[pl.ds(idx_base, _NCHUNK)], idx_e)
      copies = []
      for j in range(_NCHUNK):
        dst = rows_a.at[pl.ds(j * _CHUNK, _CHUNK)]
        copies.append(pltpu.async_copy(app_h.at[idx_a.at[j]], dst, sem))
        dst = rows_e.at[pl.ds(j * _CHUNK, _CHUNK)]
        copies.append(pltpu.async_copy(ent_h.at[idx_e.at[j]], dst, sem))
      for c in copies:
        c.wait()

      def group(g, carry):
        row_idx = g * 16 + lax.iota(jnp.int32, 16)
        accs = [jnp.zeros((16,), jnp.float32) for _ in range(4)]
        for j in range(_EMB_DIM):
          col = jnp.full((16,), j, jnp.int32)
          va = plsc.load_gather(rows_a, [row_idx, col])
          ve = plsc.load_gather(rows_e, [row_idx, col])
          accs[j % 4] = accs[j % 4] + va * ve
        svec[pl.ds(g * 16, 16)] = (accs[0] + accs[1]) + (accs[2] + accs[3])
        return carry

      lax.fori_loop(0, _ROWS_PER_W // 16, group, 0)
      pltpu.sync_copy(svec, out_h.at[pl.ds(out_base, _ROWS_PER_W)])

  return k(pos_app, pos_entity, neg_app, neg_entity, app_emb, entity_emb)


def _tc_loss_body(p_ref, n_ref, o_ref):
  p = p_ref[...]
  n = -n_ref[...]
  lp = jnp.minimum(p, 0.0) - jnp.log(1.0 + jnp.exp(-jnp.abs(p)))
  ln = jnp.minimum(n, 0.0) - jnp.log(1.0 + jnp.exp(-jnp.abs(n)))
  o_ref[0, 0] = -(jnp.sum(lp) + jnp.sum(ln))


def _tc_loss(pos_scores, neg_scores):
  out = pl.pallas_call(
      _tc_loss_body,
      out_shape=jax.ShapeDtypeStruct((1, 1), jnp.float32),
      out_specs=pl.BlockSpec(memory_space=pltpu.SMEM),
  )(pos_scores.reshape(_BATCH // 128, 128), neg_scores.reshape(_BATCH // 128, 128))
  return out[0, 0]


def kernel(pos_app, pos_entity, neg_app, neg_entity, app_emb, entity_emb):
  shape2d = (_BATCH // _IDX_COLS, _IDX_COLS)
  pos_scores, neg_scores = _sc_scores(
      pos_app.reshape(shape2d), pos_entity.reshape(shape2d),
      neg_app.reshape(shape2d), neg_entity.reshape(shape2d),
      app_emb, entity_emb)
  return _tc_loss(pos_scores, neg_scores)


# speedup vs baseline: 1.2320x; 1.2320x over previous
import functools

import jax
import jax.numpy as jnp
from jax import lax
from jax.experimental import pallas as pl
from jax.experimental.pallas import tpu as pltpu
from jax.experimental.pallas import tpu_sc as plsc

_APP_COUNT = 100000
_ENTITY_COUNT = 1000000
_EMB_DIM = 64
_BATCH = 16384

_NC = 2
_NS = 16
_NW = _NC * _NS
_ROWS_PER_W = _BATCH // _NW
_HALF = _ROWS_PER_W // 2
_CHUNK = 128
_IDX_COLS = 128
_IDX_ROWS_PER_W = _ROWS_PER_W // _IDX_COLS

_PACK_IN_COLS = 2048
_PACK_OUT_ROWS = _PACK_IN_COLS // 2


def _pack_body(i_ref, o_ref):
  t = i_ref[...]
  left = t[:, :_PACK_OUT_ROWS].T
  right = t[:, _PACK_OUT_ROWS:].T
  o_ref[...] = jnp.concatenate([left, right], axis=1)


def _pack_pairs(table_t, n):
  grid = pl.cdiv(n, _PACK_IN_COLS)
  return pl.pallas_call(
      _pack_body,
      grid=(grid,),
      in_specs=[pl.BlockSpec((_EMB_DIM, _PACK_IN_COLS), lambda i: (0, i))],
      out_specs=pl.BlockSpec((_PACK_OUT_ROWS, 128), lambda i: (i, 0)),
      out_shape=jax.ShapeDtypeStruct((grid * _PACK_OUT_ROWS, 128), jnp.float32),
  )(table_t)


def _sc_scores(pa2, pe2, na2, ne2, pa, pe, na, ne, app_p, ent_p):
  mesh = plsc.VectorSubcoreMesh(core_axis_name="c", subcore_axis_name="s")

  @functools.partial(
      pl.kernel,
      out_type=(
          jax.ShapeDtypeStruct((_BATCH,), jnp.float32),
          jax.ShapeDtypeStruct((_BATCH,), jnp.float32),
      ),
      mesh=mesh,
      compiler_params=pltpu.CompilerParams(needs_layout_passes=False),
      scratch_types=[
          pltpu.VMEM((_IDX_ROWS_PER_W, _IDX_COLS), jnp.int32),
          pltpu.VMEM((_IDX_ROWS_PER_W, _IDX_COLS), jnp.int32),
          pltpu.VMEM((_ROWS_PER_W,), jnp.int32),
          pltpu.VMEM((_ROWS_PER_W,), jnp.int32),
          pltpu.VMEM((_HALF, 128), jnp.float32),
          pltpu.VMEM((_HALF, 128), jnp.float32),
          pltpu.VMEM((_HALF,), jnp.float32),
          pltpu.SemaphoreType.DMA,
      ],
  )
  def k(pa2_h, pe2_h, na2_h, ne2_h, pa_h, pe_h, na_h, ne_h, app_h, ent_h,
        pos_out, neg_out, idx_a, idx_e, ora, ore, rows_a, rows_e, svec, sem):
    wid = lax.axis_index("s") * _NC + lax.axis_index("c")
    idx_base = wid * _IDX_ROWS_PER_W
    out_base = wid * _ROWS_PER_W

    phases = (
        (pa2_h, pe2_h, pa_h, pe_h, pos_out),
        (na2_h, ne2_h, na_h, ne_h, neg_out),
    )
    for ia2_h, ie2_h, ia_h, ie_h, out_h in phases:
      pltpu.sync_copy(ia2_h.at[pl.ds(idx_base, _IDX_ROWS_PER_W)], idx_a)
      pltpu.sync_copy(ie2_h.at[pl.ds(idx_base, _IDX_ROWS_PER_W)], idx_e)
      for r in range(_IDX_ROWS_PER_W):
        pltpu.sync_copy(ia_h.at[idx_base + r], ora.at[pl.ds(r * 128, 128)])
        pltpu.sync_copy(ie_h.at[idx_base + r], ore.at[pl.ds(r * 128, 128)])

      for half in range(2):
        copies = []
        for c in range(_HALF // _CHUNK):
          src_row = half * (_HALF // _CHUNK) + c
          dst = rows_a.at[pl.ds(c * _CHUNK, _CHUNK)]
          copies.append(pltpu.async_copy(app_h.at[idx_a.at[src_row]], dst, sem))
          dst = rows_e.at[pl.ds(c * _CHUNK, _CHUNK)]
          copies.append(pltpu.async_copy(ent_h.at[idx_e.at[src_row]], dst, sem))
        for c in copies:
          c.wait()

        def group(g, carry):
          row_idx = g * 16 + lax.iota(jnp.int32, 16)
          col_a = ((ora[pl.ds(half * _HALF + g * 16, 16)] >> 10) & 1) * 64
          col_e = ((ore[pl.ds(half * _HALF + g * 16, 16)] >> 10) & 1) * 64
          accs = [jnp.zeros((16,), jnp.float32) for _ in range(4)]
          for j in range(_EMB_DIM):
            jv = jnp.full((16,), j, jnp.int32)
            va = plsc.load_gather(rows_a, [row_idx, col_a + jv])
            ve = plsc.load_gather(rows_e, [row_idx, col_e + jv])
            accs[j % 4] = accs[j % 4] + va * ve
          svec[pl.ds(g * 16, 16)] = (accs[0] + accs[1]) + (accs[2] + accs[3])
          return carry

        lax.fori_loop(0, _HALF // 16, group, 0)
        pltpu.sync_copy(svec, out_h.at[pl.ds(out_base + half * _HALF, _HALF)])

  return k(pa2, pe2, na2, ne2, pa, pe, na, ne, app_p, ent_p)


def _tc_loss_body(p_ref, n_ref, o_ref):
  p = p_ref[...]
  n = -n_ref[...]
  lp = jnp.minimum(p, 0.0) - jnp.log(1.0 + jnp.exp(-jnp.abs(p)))
  ln = jnp.minimum(n, 0.0) - jnp.log(1.0 + jnp.exp(-jnp.abs(n)))
  o_ref[0, 0] = -(jnp.sum(lp) + jnp.sum(ln))


def _tc_loss(pos_scores, neg_scores):
  out = pl.pallas_call(
      _tc_loss_body,
      out_shape=jax.ShapeDtypeStruct((1, 1), jnp.float32),
      out_specs=pl.BlockSpec(memory_space=pltpu.SMEM),
  )(pos_scores.reshape(_BATCH // 128, 128), neg_scores.reshape(_BATCH // 128, 128))
  return out[0, 0]


def kernel(pos_app, pos_entity, neg_app, neg_entity, app_emb, entity_emb):
  app_p = _pack_pairs(app_emb.T, _APP_COUNT)
  ent_p = _pack_pairs(entity_emb.T, _ENTITY_COUNT)
  shape2d = (_BATCH // _IDX_COLS, _IDX_COLS)

  def packed_row(e):
    return (((e >> 11) << 10) | (e & 1023)).reshape(shape2d)

  pos_scores, neg_scores = _sc_scores(
      packed_row(pos_app), packed_row(pos_entity),
      packed_row(neg_app), packed_row(neg_entity),
      pos_app.reshape(shape2d), pos_entity.reshape(shape2d),
      neg_app.reshape(shape2d), neg_entity.reshape(shape2d),
      app_p, ent_p)
  return _tc_loss(pos_scores, neg_scores)


# speedup vs baseline: 2.2743x; 1.8461x over previous
import functools

import jax
import jax.numpy as jnp
from jax import lax
from jax.experimental import pallas as pl
from jax.experimental.pallas import tpu as pltpu
from jax.experimental.pallas import tpu_sc as plsc

_APP_COUNT = 100000
_ENTITY_COUNT = 1000000
_EMB_DIM = 64
_BATCH = 16384

_NC = 2
_NS = 16
_NW = _NC * _NS
_ROWS_PER_W = _BATCH // _NW
_HALF = _ROWS_PER_W // 2
_CHUNK = 128
_IDX_COLS = 128
_IDX_ROWS_PER_W = _ROWS_PER_W // _IDX_COLS

_PACK_IN_COLS = 8192
_PACK_OUT_ROWS = _PACK_IN_COLS // 2


def _pack_body(i_ref, o_ref):
  t = i_ref[...]
  v = jnp.concatenate([t[:, :_PACK_OUT_ROWS], t[:, _PACK_OUT_ROWS:]], axis=0)
  o_ref[...] = v.T


def _tail_body(i_ref, prev_ref, o_ref):
  del prev_ref
  t = i_ref[...]
  o_ref[...] = jnp.concatenate([t, t], axis=0).T


def _pack_pairs(table_t, n):
  main = n // _PACK_IN_COLS
  rem = n - main * _PACK_IN_COLS
  assert 0 < rem < _PACK_OUT_ROWS
  tailb = 128
  while tailb < rem:
    tailb *= 2
  total_rows = main * _PACK_OUT_ROWS + tailb
  out = pl.pallas_call(
      _pack_body,
      grid=(main,),
      in_specs=[pl.BlockSpec((_EMB_DIM, _PACK_IN_COLS), lambda i: (0, i))],
      out_specs=pl.BlockSpec((_PACK_OUT_ROWS, 128), lambda i: (i, 0)),
      out_shape=jax.ShapeDtypeStruct((total_rows, 128), jnp.float32),
  )(table_t)
  tail_in_blk = (main * _PACK_IN_COLS) // tailb
  tail_out_blk = (main * _PACK_OUT_ROWS) // tailb
  return pl.pallas_call(
      _tail_body,
      grid=(1,),
      in_specs=[
          pl.BlockSpec((_EMB_DIM, tailb), lambda i: (0, tail_in_blk)),
          pl.BlockSpec((tailb, 128), lambda i: (tail_out_blk, 0)),
      ],
      out_specs=pl.BlockSpec((tailb, 128), lambda i: (tail_out_blk, 0)),
      out_shape=jax.ShapeDtypeStruct((total_rows, 128), jnp.float32),
      input_output_aliases={1: 0},
  )(table_t, out)


def _sc_scores(pa2, pe2, na2, ne2, pa, pe, na, ne, app_p, ent_p):
  mesh = plsc.VectorSubcoreMesh(core_axis_name="c", subcore_axis_name="s")

  @functools.partial(
      pl.kernel,
      out_type=(
          jax.ShapeDtypeStruct((_BATCH,), jnp.float32),
          jax.ShapeDtypeStruct((_BATCH,), jnp.float32),
      ),
      mesh=mesh,
      compiler_params=pltpu.CompilerParams(needs_layout_passes=False),
      scratch_types=[
          pltpu.VMEM((_IDX_ROWS_PER_W, _IDX_COLS), jnp.int32),
          pltpu.VMEM((_IDX_ROWS_PER_W, _IDX_COLS), jnp.int32),
          pltpu.VMEM((_ROWS_PER_W,), jnp.int32),
          pltpu.VMEM((_ROWS_PER_W,), jnp.int32),
          pltpu.VMEM((_HALF, 128), jnp.float32),
          pltpu.VMEM((_HALF, 128), jnp.float32),
          pltpu.VMEM((_HALF,), jnp.float32),
          pltpu.SemaphoreType.DMA,
      ],
  )
  def k(pa2_h, pe2_h, na2_h, ne2_h, pa_h, pe_h, na_h, ne_h, app_h, ent_h,
        pos_out, neg_out, idx_a, idx_e, ora, ore, rows_a, rows_e, svec, sem):
    wid = lax.axis_index("s") * _NC + lax.axis_index("c")
    idx_base = wid * _IDX_ROWS_PER_W
    out_base = wid * _ROWS_PER_W

    phases = (
        (pa2_h, pe2_h, pa_h, pe_h, pos_out),
        (na2_h, ne2_h, na_h, ne_h, neg_out),
    )
    for ia2_h, ie2_h, ia_h, ie_h, out_h in phases:
      pltpu.sync_copy(ia2_h.at[pl.ds(idx_base, _IDX_ROWS_PER_W)], idx_a)
      pltpu.sync_copy(ie2_h.at[pl.ds(idx_base, _IDX_ROWS_PER_W)], idx_e)
      for r in range(_IDX_ROWS_PER_W):
        pltpu.sync_copy(ia_h.at[idx_base + r], ora.at[pl.ds(r * 128, 128)])
        pltpu.sync_copy(ie_h.at[idx_base + r], ore.at[pl.ds(r * 128, 128)])

      for half in range(2):
        copies = []
        for c in range(_HALF // _CHUNK):
          src_row = half * (_HALF // _CHUNK) + c
          dst = rows_a.at[pl.ds(c * _CHUNK, _CHUNK)]
          copies.append(pltpu.async_copy(app_h.at[idx_a.at[src_row]], dst, sem))
          dst = rows_e.at[pl.ds(c * _CHUNK, _CHUNK)]
          copies.append(pltpu.async_copy(ent_h.at[idx_e.at[src_row]], dst, sem))
        for c in copies:
          c.wait()

        def group(g, carry):
          row_idx = g * 16 + lax.iota(jnp.int32, 16)
          col_a = ((ora[pl.ds(half * _HALF + g * 16, 16)] >> 12) & 1) * 64
          col_e = ((ore[pl.ds(half * _HALF + g * 16, 16)] >> 12) & 1) * 64
          accs = [jnp.zeros((16,), jnp.float32) for _ in range(4)]
          for j in range(_EMB_DIM):
            jv = jnp.full((16,), j, jnp.int32)
            va = plsc.load_gather(rows_a, [row_idx, col_a + jv])
            ve = plsc.load_gather(rows_e, [row_idx, col_e + jv])
            accs[j % 4] = accs[j % 4] + va * ve
          svec[pl.ds(g * 16, 16)] = (accs[0] + accs[1]) + (accs[2] + accs[3])
          return carry

        lax.fori_loop(0, _HALF // 16, group, 0)
        pltpu.sync_copy(svec, out_h.at[pl.ds(out_base + half * _HALF, _HALF)])

  return k(pa2, pe2, na2, ne2, pa, pe, na, ne, app_p, ent_p)


def _tc_loss_body(p_ref, n_ref, o_ref):
  p = p_ref[...]
  n = -n_ref[...]
  lp = jnp.minimum(p, 0.0) - jnp.log(1.0 + jnp.exp(-jnp.abs(p)))
  ln = jnp.minimum(n, 0.0) - jnp.log(1.0 + jnp.exp(-jnp.abs(n)))
  o_ref[0, 0] = -(jnp.sum(lp) + jnp.sum(ln))


def _tc_loss(pos_scores, neg_scores):
  out = pl.pallas_call(
      _tc_loss_body,
      out_shape=jax.ShapeDtypeStruct((1, 1), jnp.float32),
      out_specs=pl.BlockSpec(memory_space=pltpu.SMEM),
  )(pos_scores.reshape(_BATCH // 128, 128), neg_scores.reshape(_BATCH // 128, 128))
  return out[0, 0]


def kernel(pos_app, pos_entity, neg_app, neg_entity, app_emb, entity_emb):
  app_p = _pack_pairs(app_emb.T, _APP_COUNT)
  ent_p = _pack_pairs(entity_emb.T, _ENTITY_COUNT)
  shape2d = (_BATCH // _IDX_COLS, _IDX_COLS)

  def packed_row(e):
    return (((e >> 13) << 12) | (e & 4095)).reshape(shape2d)

  pos_scores, neg_scores = _sc_scores(
      packed_row(pos_app), packed_row(pos_entity),
      packed_row(neg_app), packed_row(neg_entity),
      pos_app.reshape(shape2d), pos_entity.reshape(shape2d),
      neg_app.reshape(shape2d), neg_entity.reshape(shape2d),
      app_p, ent_p)
  return _tc_loss(pos_scores, neg_scores)


# speedup vs baseline: 2.5101x; 1.1037x over previous
import functools

import jax
import jax.numpy as jnp
from jax import lax
from jax.experimental import pallas as pl
from jax.experimental.pallas import tpu as pltpu
from jax.experimental.pallas import tpu_sc as plsc

_APP_COUNT = 100000
_ENTITY_COUNT = 1000000
_EMB_DIM = 64
_BATCH = 16384

_NC = 2
_NS = 16
_NW = _NC * _NS
_ROWS_PER_W = _BATCH // _NW
_HALF = _ROWS_PER_W // 2
_CHUNK = 128
_IDX_COLS = 128
_IDX_ROWS_PER_W = _ROWS_PER_W // _IDX_COLS

_PACK_IN_COLS = 16384
_PACK_OUT_ROWS = _PACK_IN_COLS // 2


def _pack_body(i_ref, o_ref):
  t = i_ref[...]
  v = jnp.concatenate([t[:, :_PACK_OUT_ROWS], t[:, _PACK_OUT_ROWS:]], axis=0)
  o_ref[...] = v.T


def _tail_body(i_ref, prev_ref, o_ref):
  del prev_ref
  t = i_ref[...]
  o_ref[...] = jnp.concatenate([t, t], axis=0).T


def _pack_pairs(table_t, n):
  main = n // _PACK_IN_COLS
  rem = n - main * _PACK_IN_COLS
  assert 0 < rem < _PACK_OUT_ROWS
  tailb = 128
  while tailb < rem:
    tailb *= 2
  total_rows = main * _PACK_OUT_ROWS + tailb
  out = pl.pallas_call(
      _pack_body,
      grid=(main,),
      in_specs=[pl.BlockSpec((_EMB_DIM, _PACK_IN_COLS), lambda i: (0, i))],
      out_specs=pl.BlockSpec((_PACK_OUT_ROWS, 128), lambda i: (i, 0)),
      out_shape=jax.ShapeDtypeStruct((total_rows, 128), jnp.float32),
  )(table_t)
  tail_in_blk = (main * _PACK_IN_COLS) // tailb
  tail_out_blk = (main * _PACK_OUT_ROWS) // tailb
  return pl.pallas_call(
      _tail_body,
      grid=(1,),
      in_specs=[
          pl.BlockSpec((_EMB_DIM, tailb), lambda i: (0, tail_in_blk)),
          pl.BlockSpec((tailb, 128), lambda i: (tail_out_blk, 0)),
      ],
      out_specs=pl.BlockSpec((tailb, 128), lambda i: (tail_out_blk, 0)),
      out_shape=jax.ShapeDtypeStruct((total_rows, 128), jnp.float32),
      input_output_aliases={1: 0},
  )(table_t, out)


def _sc_scores(pa2, pe2, na2, ne2, pa, pe, na, ne, app_p, ent_p):
  mesh = plsc.VectorSubcoreMesh(core_axis_name="c", subcore_axis_name="s")

  @functools.partial(
      pl.kernel,
      out_type=(
          jax.ShapeDtypeStruct((_BATCH,), jnp.float32),
          jax.ShapeDtypeStruct((_BATCH,), jnp.float32),
      ),
      mesh=mesh,
      compiler_params=pltpu.CompilerParams(needs_layout_passes=False),
      scratch_types=[
          pltpu.VMEM((_IDX_ROWS_PER_W, _IDX_COLS), jnp.int32),
          pltpu.VMEM((_IDX_ROWS_PER_W, _IDX_COLS), jnp.int32),
          pltpu.VMEM((_ROWS_PER_W,), jnp.int32),
          pltpu.VMEM((_ROWS_PER_W,), jnp.int32),
          pltpu.VMEM((_HALF, 128), jnp.float32),
          pltpu.VMEM((_HALF, 128), jnp.float32),
          pltpu.VMEM((_HALF,), jnp.float32),
          pltpu.SemaphoreType.DMA,
      ],
  )
  def k(pa2_h, pe2_h, na2_h, ne2_h, pa_h, pe_h, na_h, ne_h, app_h, ent_h,
        pos_out, neg_out, idx_a, idx_e, ora, ore, rows_a, rows_e, svec, sem):
    wid = lax.axis_index("s") * _NC + lax.axis_index("c")
    idx_base = wid * _IDX_ROWS_PER_W
    out_base = wid * _ROWS_PER_W

    phases = (
        (pa2_h, pe2_h, pa_h, pe_h, pos_out),
        (na2_h, ne2_h, na_h, ne_h, neg_out),
    )
    for ia2_h, ie2_h, ia_h, ie_h, out_h in phases:
      pltpu.sync_copy(ia2_h.at[pl.ds(idx_base, _IDX_ROWS_PER_W)], idx_a)
      pltpu.sync_copy(ie2_h.at[pl.ds(idx_base, _IDX_ROWS_PER_W)], idx_e)
      for r in range(_IDX_ROWS_PER_W):
        pltpu.sync_copy(ia_h.at[idx_base + r], ora.at[pl.ds(r * 128, 128)])
        pltpu.sync_copy(ie_h.at[idx_base + r], ore.at[pl.ds(r * 128, 128)])

      for half in range(2):
        copies = []
        for c in range(_HALF // _CHUNK):
          src_row = half * (_HALF // _CHUNK) + c
          dst = rows_a.at[pl.ds(c * _CHUNK, _CHUNK)]
          copies.append(pltpu.async_copy(app_h.at[idx_a.at[src_row]], dst, sem))
          dst = rows_e.at[pl.ds(c * _CHUNK, _CHUNK)]
          copies.append(pltpu.async_copy(ent_h.at[idx_e.at[src_row]], dst, sem))
        for c in copies:
          c.wait()

        def group(g, carry):
          row_idx = g * 16 + lax.iota(jnp.int32, 16)
          col_a = ((ora[pl.ds(half * _HALF + g * 16, 16)] >> 13) & 1) * 64
          col_e = ((ore[pl.ds(half * _HALF + g * 16, 16)] >> 13) & 1) * 64
          accs = [jnp.zeros((16,), jnp.float32) for _ in range(4)]
          for j in range(_EMB_DIM):
            jv = jnp.full((16,), j, jnp.int32)
            va = plsc.load_gather(rows_a, [row_idx, col_a + jv])
            ve = plsc.load_gather(rows_e, [row_idx, col_e + jv])
            accs[j % 4] = accs[j % 4] + va * ve
          svec[pl.ds(g * 16, 16)] = (accs[0] + accs[1]) + (accs[2] + accs[3])
          return carry

        lax.fori_loop(0, _HALF // 16, group, 0)
        pltpu.sync_copy(svec, out_h.at[pl.ds(out_base + half * _HALF, _HALF)])

  return k(pa2, pe2, na2, ne2, pa, pe, na, ne, app_p, ent_p)


def _tc_loss_body(p_ref, n_ref, o_ref):
  p = p_ref[...]
  n = -n_ref[...]
  lp = jnp.minimum(p, 0.0) - jnp.log(1.0 + jnp.exp(-jnp.abs(p)))
  ln = jnp.minimum(n, 0.0) - jnp.log(1.0 + jnp.exp(-jnp.abs(n)))
  o_ref[0, 0] = -(jnp.sum(lp) + jnp.sum(ln))


def _tc_loss(pos_scores, neg_scores):
  out = pl.pallas_call(
      _tc_loss_body,
      out_shape=jax.ShapeDtypeStruct((1, 1), jnp.float32),
      out_specs=pl.BlockSpec(memory_space=pltpu.SMEM),
  )(pos_scores.reshape(_BATCH // 128, 128), neg_scores.reshape(_BATCH // 128, 128))
  return out[0, 0]


def kernel(pos_app, pos_entity, neg_app, neg_entity, app_emb, entity_emb):
  app_p = _pack_pairs(app_emb.T, _APP_COUNT)
  ent_p = _pack_pairs(entity_emb.T, _ENTITY_COUNT)
  shape2d = (_BATCH // _IDX_COLS, _IDX_COLS)

  def packed_row(e):
    return (((e >> 14) << 13) | (e & 8191)).reshape(shape2d)

  pos_scores, neg_scores = _sc_scores(
      packed_row(pos_app), packed_row(pos_entity),
      packed_row(neg_app), packed_row(neg_entity),
      pos_app.reshape(shape2d), pos_entity.reshape(shape2d),
      neg_app.reshape(shape2d), neg_entity.reshape(shape2d),
      app_p, ent_p)
  return _tc_loss(pos_scores, neg_scores)


# speedup vs baseline: 2.5739x; 1.0254x over previous
import functools

import jax
import jax.numpy as jnp
from jax import lax
from jax.experimental import pallas as pl
from jax.experimental.pallas import tpu as pltpu
from jax.experimental.pallas import tpu_sc as plsc

_APP_COUNT = 100000
_ENTITY_COUNT = 1000000
_EMB_DIM = 64
_BATCH = 16384

_NC = 2
_NS = 16
_NW = _NC * _NS
_ROWS_PER_W = _BATCH // _NW
_HALF = _ROWS_PER_W // 2
_CHUNK = 128
_IDX_COLS = 128
_IDX_ROWS_PER_W = _ROWS_PER_W // _IDX_COLS

_PACK_IN_COLS = 32768
_PACK_OUT_ROWS = _PACK_IN_COLS // 2


def _pack_body(i_ref, o_ref):
  t = i_ref[...]
  v = jnp.concatenate([t[:, :_PACK_OUT_ROWS], t[:, _PACK_OUT_ROWS:]], axis=0)
  o_ref[...] = v.T


def _pack_pairs(table_t, n):
  grid = pl.cdiv(n, _PACK_IN_COLS)
  return pl.pallas_call(
      _pack_body,
      grid=(grid,),
      in_specs=[pl.BlockSpec((_EMB_DIM, _PACK_IN_COLS), lambda i: (0, i))],
      out_specs=pl.BlockSpec((_PACK_OUT_ROWS, 128), lambda i: (i, 0)),
      out_shape=jax.ShapeDtypeStruct((grid * _PACK_OUT_ROWS, 128), jnp.float32),
  )(table_t)


def _sc_scores(pa2, pe2, na2, ne2, pa, pe, na, ne, app_p, ent_p):
  mesh = plsc.VectorSubcoreMesh(core_axis_name="c", subcore_axis_name="s")

  @functools.partial(
      pl.kernel,
      out_type=(
          jax.ShapeDtypeStruct((_BATCH,), jnp.float32),
          jax.ShapeDtypeStruct((_BATCH,), jnp.float32),
      ),
      mesh=mesh,
      compiler_params=pltpu.CompilerParams(needs_layout_passes=False),
      scratch_types=[
          pltpu.VMEM((_IDX_ROWS_PER_W, _IDX_COLS), jnp.int32),
          pltpu.VMEM((_IDX_ROWS_PER_W, _IDX_COLS), jnp.int32),
          pltpu.VMEM((_ROWS_PER_W,), jnp.int32),
          pltpu.VMEM((_ROWS_PER_W,), jnp.int32),
          pltpu.VMEM((_HALF, 128), jnp.float32),
          pltpu.VMEM((_HALF, 128), jnp.float32),
          pltpu.VMEM((_HALF,), jnp.float32),
          pltpu.SemaphoreType.DMA,
      ],
  )
  def k(pa2_h, pe2_h, na2_h, ne2_h, pa_h, pe_h, na_h, ne_h, app_h, ent_h,
        pos_out, neg_out, idx_a, idx_e, ora, ore, rows_a, rows_e, svec, sem):
    wid = lax.axis_index("s") * _NC + lax.axis_index("c")
    idx_base = wid * _IDX_ROWS_PER_W
    out_base = wid * _ROWS_PER_W

    phases = (
        (pa2_h, pe2_h, pa_h, pe_h, pos_out),
        (na2_h, ne2_h, na_h, ne_h, neg_out),
    )
    for ia2_h, ie2_h, ia_h, ie_h, out_h in phases:
      pltpu.sync_copy(ia2_h.at[pl.ds(idx_base, _IDX_ROWS_PER_W)], idx_a)
      pltpu.sync_copy(ie2_h.at[pl.ds(idx_base, _IDX_ROWS_PER_W)], idx_e)
      for r in range(_IDX_ROWS_PER_W):
        pltpu.sync_copy(ia_h.at[idx_base + r], ora.at[pl.ds(r * 128, 128)])
        pltpu.sync_copy(ie_h.at[idx_base + r], ore.at[pl.ds(r * 128, 128)])

      for half in range(2):
        copies = []
        for c in range(_HALF // _CHUNK):
          src_row = half * (_HALF // _CHUNK) + c
          dst = rows_a.at[pl.ds(c * _CHUNK, _CHUNK)]
          copies.append(pltpu.async_copy(app_h.at[idx_a.at[src_row]], dst, sem))
          dst = rows_e.at[pl.ds(c * _CHUNK, _CHUNK)]
          copies.append(pltpu.async_copy(ent_h.at[idx_e.at[src_row]], dst, sem))
        for c in copies:
          c.wait()

        def group(g, carry):
          row_idx = g * 16 + lax.iota(jnp.int32, 16)
          col_a = ((ora[pl.ds(half * _HALF + g * 16, 16)] >> 14) & 1) * 64
          col_e = ((ore[pl.ds(half * _HALF + g * 16, 16)] >> 14) & 1) * 64
          accs = [jnp.zeros((16,), jnp.float32) for _ in range(4)]
          for j in range(_EMB_DIM):
            jv = jnp.full((16,), j, jnp.int32)
            va = plsc.load_gather(rows_a, [row_idx, col_a + jv])
            ve = plsc.load_gather(rows_e, [row_idx, col_e + jv])
            accs[j % 4] = accs[j % 4] + va * ve
          svec[pl.ds(g * 16, 16)] = (accs[0] + accs[1]) + (accs[2] + accs[3])
          return carry

        lax.fori_loop(0, _HALF // 16, group, 0)
        pltpu.sync_copy(svec, out_h.at[pl.ds(out_base + half * _HALF, _HALF)])

  return k(pa2, pe2, na2, ne2, pa, pe, na, ne, app_p, ent_p)


def _tc_loss_body(p_ref, n_ref, o_ref):
  p = p_ref[...]
  n = -n_ref[...]
  lp = jnp.minimum(p, 0.0) - jnp.log(1.0 + jnp.exp(-jnp.abs(p)))
  ln = jnp.minimum(n, 0.0) - jnp.log(1.0 + jnp.exp(-jnp.abs(n)))
  o_ref[0, 0] = -(jnp.sum(lp) + jnp.sum(ln))


def _tc_loss(pos_scores, neg_scores):
  out = pl.pallas_call(
      _tc_loss_body,
      out_shape=jax.ShapeDtypeStruct((1, 1), jnp.float32),
      out_specs=pl.BlockSpec(memory_space=pltpu.SMEM),
  )(pos_scores.reshape(_BATCH // 128, 128), neg_scores.reshape(_BATCH // 128, 128))
  return out[0, 0]


def kernel(pos_app, pos_entity, neg_app, neg_entity, app_emb, entity_emb):
  app_p = _pack_pairs(app_emb.T, _APP_COUNT)
  ent_p = _pack_pairs(entity_emb.T, _ENTITY_COUNT)
  shape2d = (_BATCH // _IDX_COLS, _IDX_COLS)

  def packed_row(e):
    return (((e >> 15) << 14) | (e & 16383)).reshape(shape2d)

  pos_scores, neg_scores = _sc_scores(
      packed_row(pos_app), packed_row(pos_entity),
      packed_row(neg_app), packed_row(neg_entity),
      pos_app.reshape(shape2d), pos_entity.reshape(shape2d),
      neg_app.reshape(shape2d), neg_entity.reshape(shape2d),
      app_p, ent_p)
  return _tc_loss(pos_scores, neg_scores)
